# Initial kernel scaffold; baseline (speedup 1.0000x reference)
import jax
import jax.numpy as jnp
from jax.experimental import pallas as pl


def kernel(queries, items):
    raise NotImplementedError("write your pallas kernel here")


# speedup vs baseline: 4.7607x; 4.7607x over previous
import functools

import jax
import jax.numpy as jnp
from jax.experimental import pallas as pl

_M_ITEMS = 1000
_M_PAD = 1024
_BLOCK_T = 512


def _fused_kernel(q_ref, w_ref, o_ref):
    s = jnp.dot(q_ref[...], w_ref[...], preferred_element_type=jnp.float32)
    col = jax.lax.broadcasted_iota(jnp.int32, s.shape, 1)
    s = jnp.where(col < _M_ITEMS, s, -1e30)
    m = jnp.max(s, axis=1, keepdims=True)
    denom = jnp.sum(jnp.exp(s - m), axis=1, keepdims=True)
    o_ref[...] = 1.0 / denom


@functools.partial(jax.jit, static_argnames=())
def kernel(queries, items):
    d_model = queries.shape[-1]
    q = queries.reshape(-1, d_model)
    t = q.shape[0]
    w = jnp.zeros((d_model, _M_PAD), jnp.float32).at[:, :_M_ITEMS].set(items.T)
    grid = (t // _BLOCK_T,)
    out = pl.pallas_call(
        _fused_kernel,
        grid=grid,
        in_specs=[
            pl.BlockSpec((_BLOCK_T, d_model), lambda i: (i, 0)),
            pl.BlockSpec((d_model, _M_PAD), lambda i: (0, 0)),
        ],
        out_specs=pl.BlockSpec((_BLOCK_T, 1), lambda i: (i, 0)),
        out_shape=jax.ShapeDtypeStruct((t, 1), jnp.float32),
    )(q, w)
    return out
